# baseline (speedup 1.0000x reference)
import functools

import jax
import jax.numpy as jnp
from jax import lax
from jax.experimental import pallas as pl
from jax.experimental.pallas import tpu as pltpu
from jax.experimental.pallas import tpu_sc as plsc

NUM_EXPERTS = 64
TOP_K = 8
HIDDEN = 4096
TOKENS = 32768
BLK_T = 512

_NUM_WORKERS = 32
_ROWS = TOKENS // _NUM_WORKERS
_PAIRS = _ROWS // 2


def _logits_block(h_ref, w_ref, logits_ref):
    h = h_ref[...].astype(jnp.bfloat16)
    acc = jnp.dot(h, w_ref[...], preferred_element_type=jnp.float32)
    logits_ref[...] = acc.astype(jnp.bfloat16).astype(jnp.float32)


def _tc_logits(hidden, wt):
    return pl.pallas_call(
        _logits_block,
        grid=(TOKENS // BLK_T,),
        in_specs=[
            pl.BlockSpec((BLK_T, HIDDEN), lambda i: (i, 0)),
            pl.BlockSpec((HIDDEN, NUM_EXPERTS), lambda i: (0, 0)),
        ],
        out_specs=pl.BlockSpec((BLK_T, NUM_EXPERTS), lambda i: (i, 0)),
        out_shape=jax.ShapeDtypeStruct((TOKENS, NUM_EXPERTS), jnp.float32),
        compiler_params=pltpu.CompilerParams(
            dimension_semantics=("arbitrary",),
        ),
    )(hidden, wt)


def _gather16(x, idx):
    return x.at[idx].get(mode="promise_in_bounds")


_SC_MESH = plsc.VectorSubcoreMesh(core_axis_name="c", subcore_axis_name="s")


@functools.partial(
    pl.kernel,
    mesh=_SC_MESH,
    compiler_params=pltpu.CompilerParams(needs_layout_passes=False,
                                         use_tc_tiling_on_sc=False),
    out_type=[
        jax.ShapeDtypeStruct((TOKENS // 2, 16), jnp.int32),
        jax.ShapeDtypeStruct((TOKENS // 2, 16), jnp.float32),
    ],
    scratch_types=[
        pltpu.VMEM((_ROWS, NUM_EXPERTS), jnp.float32),
        pltpu.VMEM((_PAIRS, 16), jnp.int32),
        pltpu.VMEM((_PAIRS, 16), jnp.float32),
    ],
)
def _sc_topk(logits_hbm, idx_hbm, wts_hbm, in_v, idx_v, wts_v):
    wid = lax.axis_index("s") * 2 + lax.axis_index("c")
    base = wid * _ROWS
    pltpu.sync_copy(logits_hbm.at[pl.ds(base, _ROWS)], in_v)

    lane = lax.iota(jnp.int32, 16)
    low8 = lane < 8
    lo_idx = lane & 7
    max_idx = jnp.where(low8, 0, 8)
    seven = jnp.full((16,), 7, jnp.int32)
    fifteen = jnp.full((16,), 15, jnp.int32)
    sign = jnp.uint32(0x80000000)
    himask = jnp.uint32(0xFFFF0000)

    def _sortd(k):
        return plsc.sort_key_val(k, lane, descending=True)[0]

    def token_top(tok):
        srt = []
        for j in range(4):
            v = in_v[tok, pl.ds(j * 16, 16)]
            bits = lax.bitcast_convert_type(v, jnp.uint32)
            mono = jnp.where((bits >> 31) == 1, ~bits, bits ^ sign)
            tie = (63 - (lane + 16 * j)).astype(jnp.uint32)
            srt.append(_sortd((mono & himask) | tie))

        def merge(a, b):
            return _sortd(jnp.where(low8, a, _gather16(b, lo_idx)))

        return merge(merge(srt[0], srt[1]), merge(srt[2], srt[3]))

    def body(p, carry):
        fa = token_top(2 * p)
        fb = token_top(2 * p + 1)
        pk = jnp.where(low8, fa, _gather16(fb, lo_idx))
        expert = 63 - lax.bitcast_convert_type(pk & jnp.uint32(63), jnp.int32)
        vbits = jnp.where((pk >> 31) == 1, (pk ^ sign) & himask,
                          (~pk) & himask)
        v = lax.bitcast_convert_type(vbits, jnp.float32)
        e = jnp.exp(v - _gather16(v, max_idx))
        cs = jnp.cumsum(e)
        s_a = _gather16(cs, seven)
        denom = jnp.where(low8, s_a, _gather16(cs, fifteen) - s_a)
        idx_v[p, :] = expert
        wts_v[p, :] = e / denom
        return carry

    lax.fori_loop(0, _PAIRS, body, None)

    pbase = wid * _PAIRS
    pltpu.sync_copy(idx_v, idx_hbm.at[pl.ds(pbase, _PAIRS)])
    pltpu.sync_copy(wts_v, wts_hbm.at[pl.ds(pbase, _PAIRS)])


def kernel(hidden, W):
    wt = W.astype(jnp.bfloat16).T
    logits = _tc_logits(hidden, wt)
    idx_flat, wts_flat = _sc_topk(logits)
    indices = idx_flat.reshape(TOKENS, TOP_K)
    weights = wts_flat.reshape(TOKENS, TOP_K).astype(jnp.bfloat16)
    return (indices, weights, logits)
